# baseline (speedup 1.0000x reference)
import functools

import jax
import jax.numpy as jnp
from jax import lax
from jax.experimental import pallas as pl
from jax.experimental.pallas import tpu as pltpu
from jax.experimental.pallas import tpu_sc as plsc

N_IN = 128
N_OUT = 64
K = 3
GRID_E = 0.05
BATCH = 4096
NI = 8
MARGIN = 0.01

L = 16
NBLK = BATCH // L
L1BITS = 11
L1BINS = 1 << L1BITS
L2BITS = 10
L2BINS = 1 << L2BITS
SHIFT2 = 32 - L1BITS - L2BITS
NSLOT = 8
FPW = 4
NCORES = 2
NSUB = 16

_mesh = plsc.VectorSubcoreMesh(core_axis_name="c", subcore_axis_name="s")


@functools.partial(
    pl.kernel,
    mesh=_mesh,
    out_type=jax.ShapeDtypeStruct((N_IN * N_OUT, L), jnp.float32),
    scratch_types=[
        pltpu.VMEM((FPW, BATCH), jnp.float32),
        pltpu.VMEM((BATCH,), jnp.int32),
        pltpu.VMEM((L1BINS,), jnp.int32),
        pltpu.VMEM((L1BINS,), jnp.int32),
        pltpu.VMEM((NSLOT * L2BINS,), jnp.int32),
        pltpu.VMEM((L1BINS // L,), jnp.int32),
        pltpu.VMEM((NSLOT * L2BINS // L,), jnp.int32),
        pltpu.VMEM((L,), jnp.float32),
        pltpu.VMEM((FPW, L), jnp.float32),
        pltpu.SemaphoreType.DMA,
    ],
    compiler_params=pltpu.CompilerParams(needs_layout_passes=False),
)
def _sc_grid_kernel(xt_hbm, out_hbm, vals, keys, hist, table, l2, c1, c2,
                    qbuf, grids, sem):
    wid = lax.axis_index("c") * NSUB + lax.axis_index("s")
    base_feat = wid * FPW
    pltpu.sync_copy(xt_hbm.at[pl.ds(base_feat, FPW)], vals)

    lanes = lax.broadcasted_iota(jnp.int32, (L,), 0)
    ones_i = jnp.ones((L,), jnp.int32)
    zeros_i = jnp.zeros((L,), jnp.int32)
    ranks = jnp.minimum(lanes + 1, 7) * 512
    tgt = ranks + 1

    for f in range(FPW):
        seven_i = jnp.full((L,), NSLOT - 1, jnp.int32)

        def z1(i, _):
            for j in range(8):
                hist[pl.ds((i * 8 + j) * L, L)] = zeros_i
                table[pl.ds((i * 8 + j) * L, L)] = seven_i
            return 0
        lax.fori_loop(0, L1BINS // L // 8, z1, 0)

        big = jnp.full((L,), jnp.inf, jnp.float32)

        def pa(i, carry):
            mn, mx = carry
            for j in range(8):
                v = vals[f, pl.ds((i * 8 + j) * L, L)]
                bits = lax.bitcast_convert_type(v, jnp.int32)
                key = jnp.where(bits < 0, ~bits, bits | jnp.int32(-(2 ** 31)))
                keys[pl.ds((i * 8 + j) * L, L)] = key
                bin1 = lax.shift_right_logical(key, L1BITS + L2BITS)
                plsc.addupdate_scatter(hist, [bin1], ones_i)
                mn = jnp.minimum(mn, v)
                mx = jnp.maximum(mx, v)
            return (mn, mx)
        mn_v, mx_v = lax.fori_loop(0, NBLK // 8, pa, (big, -big))
        mn = jnp.min(mn_v)
        mx = jnp.max(mx_v)

        carry1 = jnp.int32(0)
        for g in range(L1BINS // L // L):
            base = g * L * L + lanes * L
            acc = zeros_i
            for j in range(L):
                acc = acc + plsc.load_gather(hist, [base + j])
            c1[pl.ds(g * L, L)] = plsc.cumsum(acc) + carry1
            carry1 = carry1 + jnp.sum(acc)

        lo = zeros_i
        hi = jnp.full((L,), L1BINS // L, jnp.int32)
        for _ in range(8):
            mid = (lo + hi) >> 1
            cv = plsc.load_gather(c1, [jnp.minimum(mid, L1BINS // L - 1)])
            pred = cv >= tgt
            lo = jnp.where(pred, lo, mid + 1)
            hi = jnp.where(pred, mid, hi)
        blk1 = lo
        run = jnp.where(blk1 > 0,
                        plsc.load_gather(c1, [jnp.maximum(blk1 - 1, 0)]), 0)
        jidx = jnp.full((L,), -1, jnp.int32)
        prefix = zeros_i
        for j in range(L):
            hv = plsc.load_gather(hist, [blk1 * L + j])
            c = run + hv
            hit = (jidx < 0) & (c >= tgt)
            jidx = jnp.where(hit, j, jidx)
            prefix = jnp.where(hit, run, prefix)
            run = c
        b1 = blk1 * L + jnp.maximum(jidx, 0)
        t2 = tgt - prefix

        for i in range(6, -1, -1):
            plsc.store_scatter(table, [b1],
                               jnp.full((L,), i, jnp.int32),
                               mask=lanes == i)

        def z2(i, _):
            for j in range(8):
                l2[pl.ds((i * 8 + j) * L, L)] = zeros_i
            return 0
        lax.fori_loop(0, NSLOT * L2BINS // L // 8, z2, 0)


        def pc(i, _):
            for j in range(8):
                key = keys[pl.ds((i * 8 + j) * L, L)]
                bin1 = lax.shift_right_logical(key, L1BITS + L2BITS)
                s = plsc.load_gather(table, [bin1])
                sub = lax.shift_right_logical(key, SHIFT2) & (L2BINS - 1)
                plsc.addupdate_scatter(l2, [(s << L2BITS) | sub], ones_i)
            return 0
        lax.fori_loop(0, NBLK // 8, pc, 0)

        GPS = L2BINS // L // L
        for si in range(NSLOT - 1):
            carry2 = jnp.int32(0)
            for g in range(GPS):
                base = si * L2BINS + g * L * L + lanes * L
                acc = zeros_i
                for j in range(L):
                    acc = acc + plsc.load_gather(l2, [base + j])
                c2[pl.ds(si * (L2BINS // L) + g * L, L)] = (
                    plsc.cumsum(acc) + carry2)
                carry2 = carry2 + jnp.sum(acc)

        s2 = plsc.load_gather(table, [b1])
        cb2 = s2 << (L2BITS - 4)
        lo = zeros_i
        hi = jnp.full((L,), L2BINS // L, jnp.int32)
        for _ in range(7):
            mid = (lo + hi) >> 1
            cv = plsc.load_gather(
                c2, [cb2 + jnp.minimum(mid, L2BINS // L - 1)])
            pred = cv >= t2
            lo = jnp.where(pred, lo, mid + 1)
            hi = jnp.where(pred, mid, hi)
        blk2 = lo
        run = jnp.where(blk2 > 0,
                        plsc.load_gather(c2, [cb2 + jnp.maximum(blk2 - 1, 0)]),
                        0)
        fb2 = (s2 << L2BITS) | (blk2 * L)
        jidx = jnp.full((L,), -1, jnp.int32)
        for j in range(L):
            hv = plsc.load_gather(l2, [fb2 + j])
            c = run + hv
            hit = (jidx < 0) & (c >= t2)
            jidx = jnp.where(hit, j, jidx)
            run = c
        u2 = blk2 * L + jnp.maximum(jidx, 0)
        key_est = ((b1 << (L1BITS + L2BITS)) | (u2 << SHIFT2)
                   | jnp.int32(1 << (SHIFT2 - 1)))
        rbits = jnp.where(key_est < 0,
                          key_est & jnp.int32(0x7FFFFFFF), ~key_est)
        q = lax.bitcast_convert_type(rbits, jnp.float32)

        plsc.store_scatter(qbuf, [lanes + 1], q, mask=lanes < 7)
        plsc.store_scatter(qbuf, [zeros_i], jnp.full((L,), mn),
                           mask=lanes == 0)
        plsc.store_scatter(qbuf, [jnp.full((L,), NI, jnp.int32)],
                           jnp.full((L,), mx), mask=lanes == 0)
        gidx = jnp.clip(lanes - K, 0, NI)
        adaptive = plsc.load_gather(qbuf, [gidx])
        step = (mx - mn + 2 * MARGIN) * (1.0 / NI)
        uniform = gidx.astype(jnp.float32) * step + (mn - MARGIN)
        g = GRID_E * uniform + (1.0 - GRID_E) * adaptive
        g0 = GRID_E * (mn - MARGIN) + (1.0 - GRID_E) * mn
        g8 = GRID_E * (8.0 * step + (mn - MARGIN)) + (1.0 - GRID_E) * mx
        h = (g8 - g0) * (1.0 / NI)
        ext = jnp.minimum(lanes - K, 0) + jnp.maximum(lanes - (K + NI), 0)
        grids[f, :] = g + ext.astype(jnp.float32) * h

    def ostart(k, _):
        pltpu.make_async_copy(
            grids, out_hbm.at[pl.ds(k * N_IN + base_feat, FPW)], sem).start()
        return 0
    lax.fori_loop(0, N_OUT, ostart, 0)

    def odrain(k, _):
        pltpu.make_async_copy(
            grids, out_hbm.at[pl.ds(k * N_IN + base_feat, FPW)], sem).wait()
        return 0
    lax.fori_loop(0, N_OUT, odrain, 0)


def kernel(x, new_intervals):
    del new_intervals
    xt = x.T
    out16 = _sc_grid_kernel(xt)
    return out16[:, : NI + 2 * K + 1]


# speedup vs baseline: 1.1688x; 1.1688x over previous
import functools

import jax
import jax.numpy as jnp
from jax import lax
from jax.experimental import pallas as pl
from jax.experimental.pallas import tpu as pltpu
from jax.experimental.pallas import tpu_sc as plsc

N_IN = 128
N_OUT = 64
K = 3
GRID_E = 0.05
BATCH = 4096
NI = 8
MARGIN = 0.01

L = 16
NBLK = BATCH // L
L1BITS = 11
L1BINS = 1 << L1BITS
L2BITS = 10
L2BINS = 1 << L2BITS
SHIFT2 = 32 - L1BITS - L2BITS
NSLOT = 8
FPW = 4
NCORES = 2
NSUB = 16

_mesh = plsc.VectorSubcoreMesh(core_axis_name="c", subcore_axis_name="s")


@functools.partial(
    pl.kernel,
    mesh=_mesh,
    out_type=jax.ShapeDtypeStruct((N_IN * N_OUT, L), jnp.float32),
    scratch_types=[
        pltpu.VMEM((FPW, BATCH), jnp.float32),
        pltpu.VMEM((BATCH,), jnp.int32),
        pltpu.VMEM((L1BINS,), jnp.int32),
        pltpu.VMEM((L1BINS,), jnp.int32),
        pltpu.VMEM((NSLOT * L2BINS,), jnp.int32),
        pltpu.VMEM((L1BINS // L,), jnp.int32),
        pltpu.VMEM((NSLOT * L2BINS // L,), jnp.int32),
        pltpu.VMEM((L,), jnp.float32),
        pltpu.VMEM((FPW, L), jnp.float32),
        pltpu.SemaphoreType.DMA,
    ],
    compiler_params=pltpu.CompilerParams(needs_layout_passes=False),
)
def _sc_grid_kernel(xt_hbm, out_hbm, vals, keys, hist, table, l2, c1, c2,
                    qbuf, grids, sem):
    wid = lax.axis_index("c") * NSUB + lax.axis_index("s")
    base_feat = wid * FPW
    pltpu.sync_copy(xt_hbm.at[pl.ds(base_feat, FPW)], vals)

    lanes = lax.broadcasted_iota(jnp.int32, (L,), 0)
    ones_i = jnp.ones((L,), jnp.int32)
    zeros_i = jnp.zeros((L,), jnp.int32)
    ranks = jnp.minimum(lanes + 1, 7) * 512
    tgt = ranks + 1

    for f in range(FPW):
        seven_i = jnp.full((L,), NSLOT - 1, jnp.int32)

        def z1(i, _):
            for j in range(8):
                hist[pl.ds((i * 8 + j) * L, L)] = zeros_i
                table[pl.ds((i * 8 + j) * L, L)] = seven_i
            return 0
        lax.fori_loop(0, L1BINS // L // 8, z1, 0)

        big = jnp.full((L,), jnp.inf, jnp.float32)

        def pa(i, carry):
            mn, mx = carry
            for j in range(8):
                v = vals[f, pl.ds((i * 8 + j) * L, L)]
                bits = lax.bitcast_convert_type(v, jnp.int32)
                key = jnp.where(bits < 0, ~bits, bits | jnp.int32(-(2 ** 31)))
                keys[pl.ds((i * 8 + j) * L, L)] = key
                bin1 = lax.shift_right_logical(key, L1BITS + L2BITS)
                plsc.addupdate_scatter(hist, [bin1], ones_i)
                mn = jnp.minimum(mn, v)
                mx = jnp.maximum(mx, v)
            return (mn, mx)
        mn_v, mx_v = lax.fori_loop(0, NBLK // 8, pa, (big, -big))
        mn = jnp.min(mn_v)
        mx = jnp.max(mx_v)

        def cg1(g, carry1):
            base = g * L * L + lanes * L
            acc = zeros_i
            for j in range(L):
                acc = acc + plsc.load_gather(hist, [base + j])
            c1[pl.ds(g * L, L)] = plsc.cumsum(acc) + carry1
            return carry1 + jnp.sum(acc)
        lax.fori_loop(0, L1BINS // L // L, cg1, jnp.int32(0))

        lo = zeros_i
        hi = jnp.full((L,), L1BINS // L, jnp.int32)
        for _ in range(8):
            mid = (lo + hi) >> 1
            cv = plsc.load_gather(c1, [jnp.minimum(mid, L1BINS // L - 1)])
            pred = cv >= tgt
            lo = jnp.where(pred, lo, mid + 1)
            hi = jnp.where(pred, mid, hi)
        blk1 = lo
        run = jnp.where(blk1 > 0,
                        plsc.load_gather(c1, [jnp.maximum(blk1 - 1, 0)]), 0)
        jidx = jnp.full((L,), -1, jnp.int32)
        prefix = zeros_i
        for j in range(L):
            hv = plsc.load_gather(hist, [blk1 * L + j])
            c = run + hv
            hit = (jidx < 0) & (c >= tgt)
            jidx = jnp.where(hit, j, jidx)
            prefix = jnp.where(hit, run, prefix)
            run = c
        b1 = blk1 * L + jnp.maximum(jidx, 0)
        t2 = tgt - prefix

        for i in range(6, -1, -1):
            plsc.store_scatter(table, [b1],
                               jnp.full((L,), i, jnp.int32),
                               mask=lanes == i)

        def z2(i, _):
            for j in range(8):
                l2[pl.ds((i * 8 + j) * L, L)] = zeros_i
            return 0
        lax.fori_loop(0, NSLOT * L2BINS // L // 8, z2, 0)


        def pc(i, _):
            for j in range(8):
                key = keys[pl.ds((i * 8 + j) * L, L)]
                bin1 = lax.shift_right_logical(key, L1BITS + L2BITS)
                s = plsc.load_gather(table, [bin1])
                sub = lax.shift_right_logical(key, SHIFT2) & (L2BINS - 1)
                plsc.addupdate_scatter(l2, [(s << L2BITS) | sub], ones_i)
            return 0
        lax.fori_loop(0, NBLK // 8, pc, 0)

        GPS = L2BINS // L // L
        def cg2(g, carry2):
            si = g // GPS
            gi = g - si * GPS
            carry2 = jnp.where(gi == 0, 0, carry2)
            base = si * L2BINS + gi * L * L + lanes * L
            acc = zeros_i
            for j in range(L):
                acc = acc + plsc.load_gather(l2, [base + j])
            c2[pl.ds(g * L, L)] = plsc.cumsum(acc) + carry2
            return carry2 + jnp.sum(acc)
        lax.fori_loop(0, (NSLOT - 1) * GPS, cg2, jnp.int32(0))

        s2 = plsc.load_gather(table, [b1])
        cb2 = s2 << (L2BITS - 4)
        lo = zeros_i
        hi = jnp.full((L,), L2BINS // L, jnp.int32)
        for _ in range(7):
            mid = (lo + hi) >> 1
            cv = plsc.load_gather(
                c2, [cb2 + jnp.minimum(mid, L2BINS // L - 1)])
            pred = cv >= t2
            lo = jnp.where(pred, lo, mid + 1)
            hi = jnp.where(pred, mid, hi)
        blk2 = lo
        run = jnp.where(blk2 > 0,
                        plsc.load_gather(c2, [cb2 + jnp.maximum(blk2 - 1, 0)]),
                        0)
        fb2 = (s2 << L2BITS) | (blk2 * L)
        jidx = jnp.full((L,), -1, jnp.int32)
        for j in range(L):
            hv = plsc.load_gather(l2, [fb2 + j])
            c = run + hv
            hit = (jidx < 0) & (c >= t2)
            jidx = jnp.where(hit, j, jidx)
            run = c
        u2 = blk2 * L + jnp.maximum(jidx, 0)
        key_est = ((b1 << (L1BITS + L2BITS)) | (u2 << SHIFT2)
                   | jnp.int32(1 << (SHIFT2 - 1)))
        rbits = jnp.where(key_est < 0,
                          key_est & jnp.int32(0x7FFFFFFF), ~key_est)
        q = lax.bitcast_convert_type(rbits, jnp.float32)

        plsc.store_scatter(qbuf, [lanes + 1], q, mask=lanes < 7)
        plsc.store_scatter(qbuf, [zeros_i], jnp.full((L,), mn),
                           mask=lanes == 0)
        plsc.store_scatter(qbuf, [jnp.full((L,), NI, jnp.int32)],
                           jnp.full((L,), mx), mask=lanes == 0)
        gidx = jnp.clip(lanes - K, 0, NI)
        adaptive = plsc.load_gather(qbuf, [gidx])
        step = (mx - mn + 2 * MARGIN) * (1.0 / NI)
        uniform = gidx.astype(jnp.float32) * step + (mn - MARGIN)
        g = GRID_E * uniform + (1.0 - GRID_E) * adaptive
        g0 = GRID_E * (mn - MARGIN) + (1.0 - GRID_E) * mn
        g8 = GRID_E * (8.0 * step + (mn - MARGIN)) + (1.0 - GRID_E) * mx
        h = (g8 - g0) * (1.0 / NI)
        ext = jnp.minimum(lanes - K, 0) + jnp.maximum(lanes - (K + NI), 0)
        grids[f, :] = g + ext.astype(jnp.float32) * h

    def ostart(k, _):
        pltpu.make_async_copy(
            grids, out_hbm.at[pl.ds(k * N_IN + base_feat, FPW)], sem).start()
        return 0
    lax.fori_loop(0, N_OUT, ostart, 0)

    def odrain(k, _):
        pltpu.make_async_copy(
            grids, out_hbm.at[pl.ds(k * N_IN + base_feat, FPW)], sem).wait()
        return 0
    lax.fori_loop(0, N_OUT, odrain, 0)


def kernel(x, new_intervals):
    del new_intervals
    xt = x.T
    out16 = _sc_grid_kernel(xt)
    return out16[:, : NI + 2 * K + 1]
